# baseline (speedup 1.0000x reference)
import functools

import jax
import jax.numpy as jnp
from jax import lax
from jax.experimental import pallas as pl
from jax.experimental.pallas import tpu as pltpu
from jax.experimental.pallas import tpu_sc as plsc

N = 10000
E = 320000
D_IN = 128
D_H = 128
D_OUT = 64
DEPTH = 3

NC = 2
NS = 16
NW = NC * NS
EPW = E // NW
CH = 125
NCHUNK = EPW // CH
SUP = 8
NSUP = NCHUNK // SUP
ZROWS = 1000
NZW = N // ZROWS


@functools.cache
def _sc_kernels():
    mesh = plsc.VectorSubcoreMesh(
        core_axis_name="c", subcore_axis_name="s",
        num_cores=NC, num_subcores=NS)

    params = pltpu.CompilerParams(needs_layout_passes=False)

    deg = functools.partial(
        pl.kernel,
        out_type=jax.ShapeDtypeStruct((NW, 2, N), jnp.float32),
        mesh=mesh,
        compiler_params=params,
        scratch_types=[
            pltpu.VMEM((EPW,), jnp.int32),
            pltpu.VMEM((EPW,), jnp.int32),
            pltpu.VMEM((N,), jnp.float32),
            pltpu.VMEM((N,), jnp.float32),
        ],
    )(_deg_body)

    def make_conv(width):
        return functools.partial(
            pl.kernel,
            out_type=jax.ShapeDtypeStruct((NC, N, width), jnp.float32),
            mesh=mesh,
            compiler_params=params,
            scratch_types=[
                pltpu.VMEM((SUP, CH), jnp.int32),
                pltpu.VMEM((SUP, CH), jnp.int32),
                pltpu.VMEM((SUP * CH,), jnp.float32),
                pltpu.VMEM((CH, width), jnp.float32),
                pltpu.VMEM((CH, width), jnp.float32),
                pltpu.VMEM_SHARED((N, width), jnp.float32),
                pltpu.SemaphoreType.DMA,
                pltpu.SemaphoreType.DMA,
                pltpu.SemaphoreType.DMA,
                pltpu.SemaphoreType.DMA,
            ],
        )(_conv_body)

    return deg, make_conv(D_H)


def _deg_body(src_hbm, dst_hbm, out_hbm, srcv, dstv, degs, degd):
    c = lax.axis_index("c")
    s = lax.axis_index("s")
    wid = c * NS + s
    pltpu.sync_copy(src_hbm.at[wid], srcv)
    pltpu.sync_copy(dst_hbm.at[wid], dstv)

    zeros = jnp.zeros((16,), jnp.float32)

    def zbody(k, carry):
        degs[pl.ds(k * 16, 16)] = zeros
        degd[pl.ds(k * 16, 16)] = zeros
        return carry

    lax.fori_loop(0, N // 16, zbody, 0)

    ones = jnp.ones((16,), jnp.float32)

    def body(k, carry):
        si = srcv[pl.ds(k * 16, 16)]
        di = dstv[pl.ds(k * 16, 16)]
        plsc.addupdate_scatter(degs, [si], ones)
        plsc.addupdate_scatter(degd, [di], ones)
        return carry

    lax.fori_loop(0, EPW // 16, body, 0)
    pltpu.sync_copy(degs, out_hbm.at[wid, 0])
    pltpu.sync_copy(degd, out_hbm.at[wid, 1])


def _conv_body(h_hbm, src_hbm, dst_hbm, a_hbm, z_hbm, out_hbm,
               srcv, dstv, av, rows0, rows1, agg,
               gsem0, gsem1, ssem0, ssem1):
    c = lax.axis_index("c")
    s = lax.axis_index("s")
    wid = c * NS + s
    @pl.when(s < NZW)
    def _():
        pltpu.sync_copy(z_hbm, agg.at[pl.ds(s * ZROWS, ZROWS)])

    plsc.subcore_barrier()

    bufs = (rows0, rows1)
    gsems = (gsem0, gsem1)
    ssems = (ssem0, ssem1)

    def compute(j, buf):
        def edge_block(eo, carry3):
            for u in range(5):
                e = eo * 5 + u
                bc = plsc.load_gather(
                    av, [jnp.full((16,), j * CH + e, jnp.int32)])
                for k in range(D_H // 16):
                    buf[e, pl.ds(k * 16, 16)] = \
                        buf[e, pl.ds(k * 16, 16)] * bc
            return carry3

        lax.fori_loop(0, CH // 5, edge_block, 0)

    def sup(g, carry):
        pltpu.sync_copy(src_hbm.at[wid, pl.ds(g * SUP, SUP)], srcv)
        pltpu.sync_copy(dst_hbm.at[wid, pl.ds(g * SUP, SUP)], dstv)
        pltpu.sync_copy(a_hbm.at[wid, g], av)
        gd = {0: pltpu.async_copy(h_hbm.at[srcv.at[0]], bufs[0], gsems[0])}
        sd = {}
        for j in range(SUP):
            b = j % 2
            gd[j].wait()
            if j + 1 < SUP:
                if j >= 1:
                    sd[j - 1].wait()
                gd[j + 1] = pltpu.async_copy(
                    h_hbm.at[srcv.at[j + 1]], bufs[1 - b], gsems[1 - b])
            compute(j, bufs[b])
            if j < SUP - 1:
                sd[j] = pltpu.async_copy(
                    bufs[b], agg.at[dstv.at[j]], ssems[b], add=True)
            else:
                pltpu.sync_copy(bufs[b], agg.at[dstv.at[j]], add=True)
                sd[j - 1].wait()
        return carry

    lax.fori_loop(0, NSUP, sup, 0)
    plsc.subcore_barrier()

    @pl.when(s < NZW)
    def _():
        pltpu.sync_copy(agg.at[pl.ds(s * ZROWS, ZROWS)],
                        out_hbm.at[c, pl.ds(s * ZROWS, ZROWS)])


_MB = 400
_GRID_N = N // _MB


_ECOLS = DEPTH * E // N


def _norm_body(degp_ref, out_ref):
    deg = jnp.sum(degp_ref[...], axis=0)
    out_ref[...] = jnp.where(deg > 0, lax.rsqrt(deg), 0.0)


_norm_call = pl.pallas_call(
    _norm_body,
    out_shape=jax.ShapeDtypeStruct((2, N), jnp.float32),
)


def _prologue_body(x_ref, n_ref, eps_ref, mu_ref, ls_ref, xs_ref, a_ref):
    xs_ref[...] = x_ref[...] * n_ref[...]
    a_ref[...] = mu_ref[0, 0] + jnp.exp(ls_ref[0, 0]) * eps_ref[...]


_prologue_call = pl.pallas_call(
    _prologue_body,
    grid=(_GRID_N,),
    in_specs=[
        pl.BlockSpec((_MB, D_IN), lambda i: (i, 0)),
        pl.BlockSpec((_MB, 1), lambda i: (i, 0)),
        pl.BlockSpec((_MB, _ECOLS), lambda i: (i, 0)),
        pl.BlockSpec((1, 1), lambda i: (0, 0)),
        pl.BlockSpec((1, 1), lambda i: (0, 0)),
    ],
    out_specs=[
        pl.BlockSpec((_MB, D_IN), lambda i: (i, 0)),
        pl.BlockSpec((_MB, _ECOLS), lambda i: (i, 0)),
    ],
    out_shape=[
        jax.ShapeDtypeStruct((N, D_IN), jnp.float32),
        jax.ShapeDtypeStruct((N, _ECOLS), jnp.float32),
    ],
)


def _mm0_body(p0_ref, p1_ref, nin_ref, nout_ref, w_ref, b_ref, out_ref):
    agg = (p0_ref[...] + p1_ref[...]) * nin_ref[...]
    y = jnp.dot(agg, w_ref[...], preferred_element_type=jnp.float32) + b_ref[...]
    out_ref[...] = jnp.maximum(y, 0.0) * nout_ref[...]


_mm0_call = pl.pallas_call(
    _mm0_body,
    grid=(_GRID_N,),
    in_specs=[
        pl.BlockSpec((_MB, D_H), lambda i: (i, 0)),
        pl.BlockSpec((_MB, D_H), lambda i: (i, 0)),
        pl.BlockSpec((_MB, 1), lambda i: (i, 0)),
        pl.BlockSpec((_MB, 1), lambda i: (i, 0)),
        pl.BlockSpec((D_H, D_H), lambda i: (0, 0)),
        pl.BlockSpec((1, D_H), lambda i: (0, 0)),
    ],
    out_specs=pl.BlockSpec((_MB, D_H), lambda i: (i, 0)),
    out_shape=jax.ShapeDtypeStruct((N, D_H), jnp.float32),
)


def _mm1_body(p0_ref, p1_ref, nin_ref, w_ref, b_ref,
              out_ref, sum_ref, ssq_ref):
    agg = (p0_ref[...] + p1_ref[...]) * nin_ref[...]
    y = jnp.dot(agg, w_ref[...], preferred_element_type=jnp.float32) + b_ref[...]
    y = jnp.maximum(y, 0.0)
    out_ref[...] = y

    @pl.when(pl.program_id(0) == 0)
    def _():
        sum_ref[...] = jnp.zeros_like(sum_ref)
        ssq_ref[...] = jnp.zeros_like(ssq_ref)

    sum_ref[...] += jnp.sum(y, axis=0, keepdims=True)
    ssq_ref[...] += jnp.sum(y * y, axis=0, keepdims=True)


_mm1_call = pl.pallas_call(
    _mm1_body,
    grid=(_GRID_N,),
    in_specs=[
        pl.BlockSpec((_MB, D_H), lambda i: (i, 0)),
        pl.BlockSpec((_MB, D_H), lambda i: (i, 0)),
        pl.BlockSpec((_MB, 1), lambda i: (i, 0)),
        pl.BlockSpec((D_H, D_H), lambda i: (0, 0)),
        pl.BlockSpec((1, D_H), lambda i: (0, 0)),
    ],
    out_specs=[
        pl.BlockSpec((_MB, D_H), lambda i: (i, 0)),
        pl.BlockSpec((1, D_H), lambda i: (0, 0)),
        pl.BlockSpec((1, D_H), lambda i: (0, 0)),
    ],
    out_shape=[
        jax.ShapeDtypeStruct((N, D_H), jnp.float32),
        jax.ShapeDtypeStruct((1, D_H), jnp.float32),
        jax.ShapeDtypeStruct((1, D_H), jnp.float32),
    ],
)


def _bn_body(h_ref, sum_ref, ssq_ref, g_ref, bt_ref, nout_ref, out_ref):
    mu = sum_ref[...] / N
    var = ssq_ref[...] / N - mu * mu
    rstd = lax.rsqrt(var + 1e-5)
    out_ref[...] = ((h_ref[...] - mu) * rstd * g_ref[...] + bt_ref[...]) \
        * nout_ref[...]


_bn_call = pl.pallas_call(
    _bn_body,
    grid=(_GRID_N,),
    in_specs=[
        pl.BlockSpec((_MB, D_H), lambda i: (i, 0)),
        pl.BlockSpec((1, D_H), lambda i: (0, 0)),
        pl.BlockSpec((1, D_H), lambda i: (0, 0)),
        pl.BlockSpec((1, D_H), lambda i: (0, 0)),
        pl.BlockSpec((1, D_H), lambda i: (0, 0)),
        pl.BlockSpec((_MB, 1), lambda i: (i, 0)),
    ],
    out_specs=pl.BlockSpec((_MB, D_H), lambda i: (i, 0)),
    out_shape=jax.ShapeDtypeStruct((N, D_H), jnp.float32),
)


def _mm2_body(p0_ref, p1_ref, nin_ref, w_ref, b_ref, out_ref):
    agg = (p0_ref[...] + p1_ref[...]) * nin_ref[...]
    y = jnp.dot(agg, w_ref[...], preferred_element_type=jnp.float32) + b_ref[...]
    m = jnp.max(y, axis=-1, keepdims=True)
    ey = jnp.exp(y - m)
    out_ref[...] = ey / jnp.sum(ey, axis=-1, keepdims=True)


_mm2_call = pl.pallas_call(
    _mm2_body,
    grid=(_GRID_N,),
    in_specs=[
        pl.BlockSpec((_MB, D_H), lambda i: (i, 0)),
        pl.BlockSpec((_MB, D_H), lambda i: (i, 0)),
        pl.BlockSpec((_MB, 1), lambda i: (i, 0)),
        pl.BlockSpec((D_H, D_OUT), lambda i: (0, 0)),
        pl.BlockSpec((1, D_OUT), lambda i: (0, 0)),
    ],
    out_specs=pl.BlockSpec((_MB, D_OUT), lambda i: (i, 0)),
    out_shape=jax.ShapeDtypeStruct((N, D_OUT), jnp.float32),
)


def kernel(x, edge_index, eps, a_mu, a_log_sigma,
           W0, b0, W1, b1, W2, b2, bn_gamma, bn_beta):
    src = edge_index[0]
    dst = edge_index[1]
    src2 = src.reshape(NW, EPW)
    dst2 = dst.reshape(NW, EPW)
    src3 = src.reshape(NW, NCHUNK, CH)
    dst3 = dst.reshape(NW, NCHUNK, CH)

    _deg_kernel, _conv_kernel = _sc_kernels()
    degp = _deg_kernel(src2, dst2)
    nrm = _norm_call(degp)
    nout = nrm[0].reshape(N, 1)
    nin = nrm[1].reshape(N, 1)
    xs, a3 = _prologue_call(
        x, nout, eps.reshape(N, _ECOLS),
        a_mu.reshape(1, 1), a_log_sigma.reshape(1, 1))
    a3 = a3.reshape(DEPTH, NW, NSUP, SUP * CH)

    zrows = jnp.zeros((ZROWS, D_H), jnp.float32)
    b0r = b0.reshape(1, D_H)
    b1r = b1.reshape(1, D_H)
    b2r = b2.reshape(1, D_OUT)

    p = _conv_kernel(xs, src3, dst3, a3[0], zrows)
    h1s = _mm0_call(p[0], p[1], nin, nout, W0, b0r)
    p = _conv_kernel(h1s, src3, dst3, a3[1], zrows)
    h2, sums, ssq = _mm1_call(p[0], p[1], nin, W1, b1r)
    hbn = _bn_call(h2, sums, ssq, bn_gamma.reshape(1, D_H),
                   bn_beta.reshape(1, D_H), nout)
    p = _conv_kernel(hbn, src3, dst3, a3[2], zrows)
    probs = _mm2_call(p[0], p[1], nin, W2, b2r)
    return probs


# speedup vs baseline: 1.0633x; 1.0633x over previous
import functools

import jax
import jax.numpy as jnp
from jax import lax
from jax.experimental import pallas as pl
from jax.experimental.pallas import tpu as pltpu
from jax.experimental.pallas import tpu_sc as plsc

N = 10000
E = 320000
D_IN = 128
D_H = 128
D_OUT = 64
DEPTH = 3

NC = 2
NS = 16
NW = NC * NS
EPW = E // NW
CH = 125
NCHUNK = EPW // CH
SUP = 8
NSUP = NCHUNK // SUP
ZROWS = 1000
NZW = N // ZROWS


@functools.cache
def _sc_kernels():
    mesh = plsc.VectorSubcoreMesh(
        core_axis_name="c", subcore_axis_name="s",
        num_cores=NC, num_subcores=NS)

    params = pltpu.CompilerParams(needs_layout_passes=False)

    deg = functools.partial(
        pl.kernel,
        out_type=jax.ShapeDtypeStruct((NW, 2, N), jnp.float32),
        mesh=mesh,
        compiler_params=params,
        scratch_types=[
            pltpu.VMEM((EPW,), jnp.int32),
            pltpu.VMEM((EPW,), jnp.int32),
            pltpu.VMEM((N,), jnp.float32),
            pltpu.VMEM((N,), jnp.float32),
        ],
    )(_deg_body)

    def make_conv(width):
        return functools.partial(
            pl.kernel,
            out_type=jax.ShapeDtypeStruct((NC, N, width), jnp.float32),
            mesh=mesh,
            compiler_params=params,
            scratch_types=[
                pltpu.VMEM((2, SUP, CH), jnp.int32),
                pltpu.VMEM((2, SUP, CH), jnp.int32),
                pltpu.VMEM((2, SUP, CH), jnp.float32),
                pltpu.VMEM((CH, width), jnp.float32),
                pltpu.VMEM((CH, width), jnp.float32),
                pltpu.VMEM_SHARED((N, width), jnp.float32),
                pltpu.SemaphoreType.DMA,
                pltpu.SemaphoreType.DMA,
                pltpu.SemaphoreType.DMA,
                pltpu.SemaphoreType.DMA,
                pltpu.SemaphoreType.DMA,
                pltpu.SemaphoreType.DMA,
                pltpu.SemaphoreType.DMA,
            ],
        )(_conv_body)

    return deg, make_conv(D_H)


def _deg_body(src_hbm, dst_hbm, out_hbm, srcv, dstv, degs, degd):
    c = lax.axis_index("c")
    s = lax.axis_index("s")
    wid = c * NS + s
    pltpu.sync_copy(src_hbm.at[wid], srcv)
    pltpu.sync_copy(dst_hbm.at[wid], dstv)

    zeros = jnp.zeros((16,), jnp.float32)

    def zbody(k, carry):
        degs[pl.ds(k * 16, 16)] = zeros
        degd[pl.ds(k * 16, 16)] = zeros
        return carry

    lax.fori_loop(0, N // 16, zbody, 0)

    ones = jnp.ones((16,), jnp.float32)

    def body(k, carry):
        si = srcv[pl.ds(k * 16, 16)]
        di = dstv[pl.ds(k * 16, 16)]
        plsc.addupdate_scatter(degs, [si], ones)
        plsc.addupdate_scatter(degd, [di], ones)
        return carry

    lax.fori_loop(0, EPW // 16, body, 0)
    pltpu.sync_copy(degs, out_hbm.at[wid, 0])
    pltpu.sync_copy(degd, out_hbm.at[wid, 1])


def _conv_body(h_hbm, src_hbm, dst_hbm, a_hbm, z_hbm, out_hbm,
               srcv, dstv, av, rows0, rows1, agg,
               gsem0, gsem1, ssem0, ssem1, isems, isemd, isema):
    c = lax.axis_index("c")
    s = lax.axis_index("s")
    wid = c * NS + s
    @pl.when(s < NZW)
    def _():
        pltpu.sync_copy(z_hbm, agg.at[pl.ds(s * ZROWS, ZROWS)])

    def stage(g, p):
        return (
            pltpu.async_copy(
                src_hbm.at[wid, pl.ds(g * SUP, SUP)], srcv.at[p], isems),
            pltpu.async_copy(
                dst_hbm.at[wid, pl.ds(g * SUP, SUP)], dstv.at[p], isemd),
            pltpu.async_copy(a_hbm.at[wid, g], av.at[p], isema),
        )

    for t in stage(0, 0):
        t.wait()
    plsc.subcore_barrier()

    bufs = (rows0, rows1)
    gsems = (gsem0, gsem1)
    ssems = (ssem0, ssem1)

    def compute(avj, buf):
        def edge_block(eo, carry3):
            for u in range(5):
                e = eo * 5 + u
                bc = plsc.load_gather(avj, [jnp.full((16,), e, jnp.int32)])
                for k in range(D_H // 16):
                    buf[e, pl.ds(k * 16, 16)] = \
                        buf[e, pl.ds(k * 16, 16)] * bc
            return carry3

        lax.fori_loop(0, CH // 5, edge_block, 0)

    def process(p):
        gd = {0: pltpu.async_copy(h_hbm.at[srcv.at[p, 0]], bufs[0], gsems[0])}
        sd = {}
        for j in range(SUP):
            b = j % 2
            gd[j].wait()
            if j + 1 < SUP:
                if j >= 1:
                    sd[j - 1].wait()
                gd[j + 1] = pltpu.async_copy(
                    h_hbm.at[srcv.at[p, j + 1]], bufs[1 - b], gsems[1 - b])
            compute(av.at[p, j], bufs[b])
            if j < SUP - 1:
                sd[j] = pltpu.async_copy(
                    bufs[b], agg.at[dstv.at[p, j]], ssems[b], add=True)
            else:
                pltpu.sync_copy(bufs[b], agg.at[dstv.at[p, j]], add=True)
                sd[j - 1].wait()

    def sup2(gg, carry):
        g0 = gg * 2
        d1 = stage(g0 + 1, 1)
        process(0)
        for t in d1:
            t.wait()
        d2 = stage(jnp.minimum(g0 + 2, NSUP - 1), 0)
        process(1)
        for t in d2:
            t.wait()
        return carry

    lax.fori_loop(0, NSUP // 2, sup2, 0)
    plsc.subcore_barrier()

    @pl.when(s < NZW)
    def _():
        pltpu.sync_copy(agg.at[pl.ds(s * ZROWS, ZROWS)],
                        out_hbm.at[c, pl.ds(s * ZROWS, ZROWS)])


_MB = 400
_GRID_N = N // _MB


def _norm_body(degp_ref, out_ref):
    deg = jnp.sum(degp_ref[...], axis=0)
    out_ref[...] = jnp.where(deg > 0, lax.rsqrt(deg), 0.0)


_norm_call = pl.pallas_call(
    _norm_body,
    out_shape=jax.ShapeDtypeStruct((2, N), jnp.float32),
)


def _a_body(eps_ref, mu_ref, ls_ref, out_ref):
    out_ref[...] = mu_ref[0, 0] + jnp.exp(ls_ref[0, 0]) * eps_ref[...]


_a_call = pl.pallas_call(
    _a_body,
    out_shape=jax.ShapeDtypeStruct((DEPTH * E // 128, 128), jnp.float32),
)


def _scale_body(x_ref, n_ref, out_ref):
    out_ref[...] = x_ref[...] * n_ref[...]


_scale_call = pl.pallas_call(
    _scale_body,
    grid=(_GRID_N,),
    in_specs=[
        pl.BlockSpec((_MB, D_IN), lambda i: (i, 0)),
        pl.BlockSpec((_MB, 1), lambda i: (i, 0)),
    ],
    out_specs=pl.BlockSpec((_MB, D_IN), lambda i: (i, 0)),
    out_shape=jax.ShapeDtypeStruct((N, D_IN), jnp.float32),
)


def _mm0_body(p0_ref, p1_ref, nin_ref, nout_ref, w_ref, b_ref, out_ref):
    agg = (p0_ref[...] + p1_ref[...]) * nin_ref[...]
    y = jnp.dot(agg, w_ref[...], preferred_element_type=jnp.float32) + b_ref[...]
    out_ref[...] = jnp.maximum(y, 0.0) * nout_ref[...]


_mm0_call = pl.pallas_call(
    _mm0_body,
    grid=(_GRID_N,),
    in_specs=[
        pl.BlockSpec((_MB, D_H), lambda i: (i, 0)),
        pl.BlockSpec((_MB, D_H), lambda i: (i, 0)),
        pl.BlockSpec((_MB, 1), lambda i: (i, 0)),
        pl.BlockSpec((_MB, 1), lambda i: (i, 0)),
        pl.BlockSpec((D_H, D_H), lambda i: (0, 0)),
        pl.BlockSpec((1, D_H), lambda i: (0, 0)),
    ],
    out_specs=pl.BlockSpec((_MB, D_H), lambda i: (i, 0)),
    out_shape=jax.ShapeDtypeStruct((N, D_H), jnp.float32),
)


def _mm1_body(p0_ref, p1_ref, nin_ref, w_ref, b_ref,
              out_ref, sum_ref, ssq_ref):
    agg = (p0_ref[...] + p1_ref[...]) * nin_ref[...]
    y = jnp.dot(agg, w_ref[...], preferred_element_type=jnp.float32) + b_ref[...]
    y = jnp.maximum(y, 0.0)
    out_ref[...] = y

    @pl.when(pl.program_id(0) == 0)
    def _():
        sum_ref[...] = jnp.zeros_like(sum_ref)
        ssq_ref[...] = jnp.zeros_like(ssq_ref)

    sum_ref[...] += jnp.sum(y, axis=0, keepdims=True)
    ssq_ref[...] += jnp.sum(y * y, axis=0, keepdims=True)


_mm1_call = pl.pallas_call(
    _mm1_body,
    grid=(_GRID_N,),
    in_specs=[
        pl.BlockSpec((_MB, D_H), lambda i: (i, 0)),
        pl.BlockSpec((_MB, D_H), lambda i: (i, 0)),
        pl.BlockSpec((_MB, 1), lambda i: (i, 0)),
        pl.BlockSpec((D_H, D_H), lambda i: (0, 0)),
        pl.BlockSpec((1, D_H), lambda i: (0, 0)),
    ],
    out_specs=[
        pl.BlockSpec((_MB, D_H), lambda i: (i, 0)),
        pl.BlockSpec((1, D_H), lambda i: (0, 0)),
        pl.BlockSpec((1, D_H), lambda i: (0, 0)),
    ],
    out_shape=[
        jax.ShapeDtypeStruct((N, D_H), jnp.float32),
        jax.ShapeDtypeStruct((1, D_H), jnp.float32),
        jax.ShapeDtypeStruct((1, D_H), jnp.float32),
    ],
)


def _bn_body(h_ref, sum_ref, ssq_ref, g_ref, bt_ref, nout_ref, out_ref):
    mu = sum_ref[...] / N
    var = ssq_ref[...] / N - mu * mu
    rstd = lax.rsqrt(var + 1e-5)
    out_ref[...] = ((h_ref[...] - mu) * rstd * g_ref[...] + bt_ref[...]) \
        * nout_ref[...]


_bn_call = pl.pallas_call(
    _bn_body,
    grid=(_GRID_N,),
    in_specs=[
        pl.BlockSpec((_MB, D_H), lambda i: (i, 0)),
        pl.BlockSpec((1, D_H), lambda i: (0, 0)),
        pl.BlockSpec((1, D_H), lambda i: (0, 0)),
        pl.BlockSpec((1, D_H), lambda i: (0, 0)),
        pl.BlockSpec((1, D_H), lambda i: (0, 0)),
        pl.BlockSpec((_MB, 1), lambda i: (i, 0)),
    ],
    out_specs=pl.BlockSpec((_MB, D_H), lambda i: (i, 0)),
    out_shape=jax.ShapeDtypeStruct((N, D_H), jnp.float32),
)


def _mm2_body(p0_ref, p1_ref, nin_ref, w_ref, b_ref, out_ref):
    agg = (p0_ref[...] + p1_ref[...]) * nin_ref[...]
    y = jnp.dot(agg, w_ref[...], preferred_element_type=jnp.float32) + b_ref[...]
    m = jnp.max(y, axis=-1, keepdims=True)
    ey = jnp.exp(y - m)
    out_ref[...] = ey / jnp.sum(ey, axis=-1, keepdims=True)


_mm2_call = pl.pallas_call(
    _mm2_body,
    grid=(_GRID_N,),
    in_specs=[
        pl.BlockSpec((_MB, D_H), lambda i: (i, 0)),
        pl.BlockSpec((_MB, D_H), lambda i: (i, 0)),
        pl.BlockSpec((_MB, 1), lambda i: (i, 0)),
        pl.BlockSpec((D_H, D_OUT), lambda i: (0, 0)),
        pl.BlockSpec((1, D_OUT), lambda i: (0, 0)),
    ],
    out_specs=pl.BlockSpec((_MB, D_OUT), lambda i: (i, 0)),
    out_shape=jax.ShapeDtypeStruct((N, D_OUT), jnp.float32),
)


def kernel(x, edge_index, eps, a_mu, a_log_sigma,
           W0, b0, W1, b1, W2, b2, bn_gamma, bn_beta):
    src = edge_index[0]
    dst = edge_index[1]
    src2 = src.reshape(NW, EPW)
    dst2 = dst.reshape(NW, EPW)
    src3 = src.reshape(NW, NCHUNK, CH)
    dst3 = dst.reshape(NW, NCHUNK, CH)

    _deg_kernel, _conv_kernel = _sc_kernels()
    degp = _deg_kernel(src2, dst2)
    nrm = _norm_call(degp)
    nout = nrm[0].reshape(N, 1)
    nin = nrm[1].reshape(N, 1)
    a3 = _a_call(eps.reshape(DEPTH * E // 128, 128),
                 a_mu.reshape(1, 1), a_log_sigma.reshape(1, 1))
    a3 = a3.reshape(DEPTH, NW, NSUP, SUP, CH)
    xs = _scale_call(x, nout)

    zrows = jnp.zeros((ZROWS, D_H), jnp.float32)
    b0r = b0.reshape(1, D_H)
    b1r = b1.reshape(1, D_H)
    b2r = b2.reshape(1, D_OUT)

    p = _conv_kernel(xs, src3, dst3, a3[0], zrows)
    h1s = _mm0_call(p[0], p[1], nin, nout, W0, b0r)
    p = _conv_kernel(h1s, src3, dst3, a3[1], zrows)
    h2, sums, ssq = _mm1_call(p[0], p[1], nin, W1, b1r)
    hbn = _bn_call(h2, sums, ssq, bn_gamma.reshape(1, D_H),
                   bn_beta.reshape(1, D_H), nout)
    p = _conv_kernel(hbn, src3, dst3, a3[2], zrows)
    probs = _mm2_call(p[0], p[1], nin, W2, b2r)
    return probs
